# baseline (speedup 1.0000x reference)
import functools

import jax
import jax.numpy as jnp
from jax import lax
from jax.experimental import pallas as pl
from jax.experimental.pallas import tpu as pltpu
from jax.experimental.pallas import tpu_sc as plsc

B = 4096
L = 200
D = 64
LP = L + 2
NC, NS = 2, 16
NW = NC * NS
SEQ_PER_W = B // NW
C = 4
HALF = L // 2


def _make_gather():
    mesh = plsc.VectorSubcoreMesh(core_axis_name="c", subcore_axis_name="s")

    @functools.partial(
        pl.kernel,
        out_type=jax.ShapeDtypeStruct((B * LP, D), jnp.float32),
        mesh=mesh,
        scratch_types=[
            pltpu.VMEM((2 * C, HALF), jnp.int32),
            pltpu.VMEM((C * LP, D), jnp.float32),
            pltpu.SemaphoreType.DMA,
        ],
        compiler_params=pltpu.CompilerParams(use_tc_tiling_on_sc=False),
    )
    def gather_kernel(idx_hbm, table_hbm, out_hbm, idx_v, pad_v, sem):
        wid = lax.axis_index("s") * NC + lax.axis_index("c")

        zeros = jnp.zeros((16,), jnp.float32)
        for c in range(C):
            for r in (c * LP, c * LP + L + 1):
                for j in range(D // 16):
                    pad_v[r, pl.ds(j * 16, 16)] = zeros

        def chunk(g, carry):
            seq0 = wid * SEQ_PER_W + g * C
            pltpu.sync_copy(idx_hbm.at[pl.ds(2 * seq0, 2 * C)], idx_v)
            copies = []
            for c in range(C):
                for h in range(2):
                    copies.append(pltpu.async_copy(
                        table_hbm.at[idx_v.at[2 * c + h]],
                        pad_v.at[pl.ds(c * LP + 1 + h * HALF, HALF)],
                        sem))
            for cp in copies:
                cp.wait()
            pltpu.sync_copy(pad_v, out_hbm.at[pl.ds(seq0 * LP, C * LP)])
            return carry

        lax.fori_loop(0, SEQ_PER_W // C, chunk, 0)

    return gather_kernel


_gather = _make_gather()


def kernel(indices, table):
    idx_rows = indices.reshape(B * L // HALF, HALF)
    out_flat = _gather(idx_rows, table)
    val_inp = out_flat.reshape(B, LP, D)
    val_len = jnp.full((B,), LP, dtype=jnp.int32)
    return val_inp, val_len


# speedup vs baseline: 1.0178x; 1.0178x over previous
import functools

import jax
import jax.numpy as jnp
from jax import lax
from jax.experimental import pallas as pl
from jax.experimental.pallas import tpu as pltpu
from jax.experimental.pallas import tpu_sc as plsc

B = 4096
L = 200
D = 64
LP = L + 2
NC, NS = 2, 16
NW = NC * NS
SEQ_PER_W = B // NW
C = 4
G = SEQ_PER_W // C
HALF = L // 2


def _make_gather():
    mesh = plsc.VectorSubcoreMesh(core_axis_name="c", subcore_axis_name="s")

    @functools.partial(
        pl.kernel,
        out_type=jax.ShapeDtypeStruct((B * LP, D), jnp.float32),
        mesh=mesh,
        scratch_types=[
            pltpu.VMEM((2 * C, HALF), jnp.int32),
            pltpu.VMEM((C * LP, D), jnp.float32),
            pltpu.VMEM((C * LP, D), jnp.float32),
            pltpu.SemaphoreType.DMA,
            pltpu.SemaphoreType.DMA,
            pltpu.SemaphoreType.DMA,
        ],
        compiler_params=pltpu.CompilerParams(use_tc_tiling_on_sc=False),
    )
    def gather_kernel(idx_hbm, table_hbm, out_hbm,
                      idx_v, pad0, pad1, gsem, wsem0, wsem1):
        wid = lax.axis_index("s") * NC + lax.axis_index("c")
        pads = (pad0, pad1)
        wsems = (wsem0, wsem1)

        zeros = jnp.zeros((16,), jnp.float32)
        for pv in pads:
            for c in range(C):
                for r in (c * LP, c * LP + L + 1):
                    for j in range(D // 16):
                        pv[r, pl.ds(j * 16, 16)] = zeros

        def step(g, b):
            pv = pads[b]
            seq0 = wid * SEQ_PER_W + g * C
            pltpu.sync_copy(idx_hbm.at[pl.ds(2 * seq0, 2 * C)], idx_v)
            copies = [
                pltpu.async_copy(
                    table_hbm.at[idx_v.at[r]],
                    pv.at[pl.ds((r // 2) * LP + 1 + (r % 2) * HALF, HALF)],
                    gsem)
                for r in range(2 * C)
            ]
            for cp in copies:
                cp.wait()
            pltpu.async_copy(pv, out_hbm.at[pl.ds(seq0 * LP, C * LP)], wsems[b])

        def drain(b):
            pltpu.make_async_copy(
                pads[b], out_hbm.at[pl.ds(0, C * LP)], wsems[b]).wait()

        def body(h, carry):
            for b in range(2):
                @pl.when(h >= 1)
                def _():
                    drain(b)
                step(2 * h + b, b)
            return carry

        lax.fori_loop(0, G // 2, body, 0)
        drain(0)
        drain(1)

    return gather_kernel


_gather = _make_gather()


def kernel(indices, table):
    idx_rows = indices.reshape(B * L // HALF, HALF)
    out_flat = _gather(idx_rows, table)
    val_inp = out_flat.reshape(B, LP, D)
    val_len = jnp.full((B,), LP, dtype=jnp.int32)
    return val_inp, val_len
